# baseline (speedup 1.0000x reference)
import dataclasses
import functools

import jax
import jax.numpy as jnp
from jax import lax
from jax.experimental import pallas as pl
from jax.experimental.pallas import tpu as pltpu
from jax.experimental.pallas import tpu_sc as plsc

N = 10000
D = 128
E = 320000
NC, NS = 2, 16
NT = NC * NS
CHUNK = 112
K = 90
E_PAD = NT * K * CHUNK
IDXB = 6
NB = K // IDXB
N_ROWS = 10112
RPT = N_ROWS // NS

_mesh = plsc.VectorSubcoreMesh(core_axis_name="core", subcore_axis_name="subcore")

_sc_params = pltpu.CompilerParams(use_tc_tiling_on_sc=False)
if "needs_layout_passes" in pltpu.CompilerParams.__dataclass_fields__:
    _sc_params = dataclasses.replace(_sc_params, needs_layout_passes=False)


def _zero_rows(buf, nrows):
    z = jnp.zeros((16,), jnp.float32)

    @pl.loop(0, nrows)
    def _(r):
        for c in range(D // 16):
            buf[r, pl.ds(c * 16, 16)] = z


@functools.partial(
    pl.kernel,
    out_type=jax.ShapeDtypeStruct((NT, N_ROWS), jnp.float32),
    mesh=_mesh,
    compiler_params=_sc_params,
    scratch_types=[
        pltpu.VMEM((K, CHUNK), jnp.int32),
        pltpu.VMEM((N_ROWS,), jnp.float32),
    ],
)
def _sc_degree(eidx_hbm, deg_hbm, idx_v, hist_v):
    c = lax.axis_index("core")
    s = lax.axis_index("subcore")
    t = c * NS + s

    z = jnp.zeros((16,), jnp.float32)

    @pl.loop(0, N_ROWS // 16)
    def _(i):
        hist_v[pl.ds(i * 16, 16)] = z

    pltpu.sync_copy(eidx_hbm.at[1].at[t], idx_v)

    ones = jnp.ones((16,), jnp.float32)

    @pl.loop(0, K)
    def _(k):
        for cc in range(CHUNK // 16):
            v = idx_v[k, pl.ds(cc * 16, 16)]
            plsc.addupdate_scatter(hist_v, [v], ones)

    pltpu.sync_copy(hist_v, deg_hbm.at[t])


NBUF = 3
PF = 2
NW = -(-RPT // CHUNK)
WREM = RPT - (NW - 1) * CHUNK


@functools.partial(
    pl.kernel,
    out_type=jax.ShapeDtypeStruct((NC, N_ROWS, D), jnp.float32),
    mesh=_mesh,
    compiler_params=_sc_params,
    scratch_types=[
        pltpu.VMEM((NBUF - 1, IDXB, CHUNK), jnp.int32),
        pltpu.VMEM((NBUF - 1, IDXB, CHUNK), jnp.int32),
        pltpu.VMEM((NBUF, CHUNK, D), jnp.float32),
        pltpu.VMEM_SHARED((N_ROWS, D), jnp.float32),
        pltpu.SemaphoreType.DMA((NBUF,)),
        pltpu.SemaphoreType.DMA((NBUF,)),
        pltpu.SemaphoreType.DMA((2,)),
    ],
)
def _sc_messages(y_hbm, eidx_hbm, acc_hbm, src_v, dst_v, gbuf, acc_sh,
                 gsem, ssem, isem):
    c = lax.axis_index("core")
    s = lax.axis_index("subcore")
    t = c * NS + s
    base = s * RPT

    def iload(blk, buf):
        pltpu.async_copy(eidx_hbm.at[0].at[t].at[pl.ds(blk * IDXB, IDXB)],
                         src_v.at[buf], isem.at[buf])
        pltpu.async_copy(eidx_hbm.at[1].at[t].at[pl.ds(blk * IDXB, IDXB)],
                         dst_v.at[buf], isem.at[buf])

    def iwait(buf):
        pltpu.make_async_copy(eidx_hbm.at[0].at[t].at[pl.ds(0, IDXB)],
                              src_v.at[buf], isem.at[buf]).wait()
        pltpu.make_async_copy(eidx_hbm.at[1].at[t].at[pl.ds(0, IDXB)],
                              dst_v.at[buf], isem.at[buf]).wait()

    _zero_rows(gbuf.at[0], CHUNK)
    iload(0, 0)
    for i in range(NW):
        sz = CHUNK if i < NW - 1 else WREM
        pltpu.async_copy(gbuf.at[0].at[pl.ds(0, sz)],
                         acc_sh.at[pl.ds(base + i * CHUNK, sz)], ssem.at[0])
    for i in range(NW):
        sz = CHUNK if i < NW - 1 else WREM
        pltpu.make_async_copy(gbuf.at[0].at[pl.ds(0, sz)],
                              acc_sh.at[pl.ds(base, sz)], ssem.at[0]).wait()
    iwait(0)
    plsc.subcore_barrier()

    def gstart(b, ibuf, j):
        pltpu.async_copy(y_hbm.at[src_v.at[ibuf].at[j]], gbuf.at[b],
                         gsem.at[b])

    def gwait(b):
        pltpu.make_async_copy(y_hbm.at[src_v.at[0].at[0]], gbuf.at[b],
                              gsem.at[b]).wait()

    def sstart(b, ibuf, j):
        pltpu.async_copy(gbuf.at[b], acc_sh.at[dst_v.at[ibuf].at[j]],
                         ssem.at[b], add=True)

    def swait(b):
        pltpu.make_async_copy(gbuf.at[b], acc_sh.at[dst_v.at[0].at[0]],
                              ssem.at[b]).wait()

    gstart(0, 0, 0)
    gstart(1, 0, 1)

    @pl.loop(0, NB)
    def _(g):
        k0 = g * IDXB
        cur = g % 2
        nxt = 1 - cur
        for j in range(IDXB):
            k = k0 + j
            b = j % NBUF
            b2 = (j + PF) % NBUF

            gwait(b)
            sstart(b, cur, j)

            if j == 1:
                @pl.when(g < NB - 1)
                def _():
                    iload(g + 1, nxt)

            if j == 4:
                @pl.when(g < NB - 1)
                def _():
                    iwait(nxt)

            @pl.when(k + PF < K)
            def _():
                @pl.when(k - 1 >= 0)
                def _():
                    swait(b2)
                if j + PF < IDXB:
                    gstart(b2, cur, j + PF)
                else:
                    gstart(b2, nxt, j + PF - IDXB)

    for j in range(NBUF):
        swait((K - NBUF + j) % NBUF)

    plsc.subcore_barrier()

    for i in range(NW):
        sz = CHUNK if i < NW - 1 else WREM
        b = i % NBUF
        if i >= NBUF:
            psz = CHUNK
            pltpu.make_async_copy(gbuf.at[b].at[pl.ds(0, psz)],
                                  acc_hbm.at[c].at[pl.ds(base, psz)],
                                  ssem.at[b]).wait()
        pltpu.sync_copy(acc_sh.at[pl.ds(base + i * CHUNK, sz)],
                        gbuf.at[b].at[pl.ds(0, sz)])
        pltpu.async_copy(gbuf.at[b].at[pl.ds(0, sz)],
                         acc_hbm.at[c].at[pl.ds(base + i * CHUNK, sz)],
                         ssem.at[b])
    for j in range(NBUF):
        i = NW - NBUF + j
        sz = CHUNK if i < NW - 1 else WREM
        pltpu.make_async_copy(gbuf.at[i % NBUF].at[pl.ds(0, sz)],
                              acc_hbm.at[c].at[pl.ds(base, sz)],
                              ssem.at[i % NBUF]).wait()


_G1 = 8
_MB = N_ROWS // _G1
_G2 = 10
_OB = N // _G2


def _mm_body(x_ref, w_ref, o_ref):
    o_ref[...] = jnp.dot(x_ref[...], w_ref[...],
                         preferred_element_type=jnp.float32,
                         precision=lax.Precision.HIGHEST)


def _tc_matmul(x, w):
    return pl.pallas_call(
        _mm_body,
        out_shape=jax.ShapeDtypeStruct((N_ROWS, D), jnp.float32),
    )(x, w)


def _dinv_col(degp):
    ones = jnp.ones((NT, 1), jnp.float32)
    deg = lax.dot_general(degp, ones, (((0,), (0,)), ((), ())),
                          preferred_element_type=jnp.float32,
                          precision=lax.Precision.HIGHEST) + 1.0
    return lax.rsqrt(deg)


def _scale_body(xw_ref, degp_ref, y_ref):
    y_ref[...] = xw_ref[...] * _dinv_col(degp_ref[...])


def _tc_scale(xw, degp):
    return pl.pallas_call(
        _scale_body,
        out_shape=jax.ShapeDtypeStruct((N_ROWS, D), jnp.float32),
    )(xw, degp)


def _mid_body(acc_ref, y_ref, degp_ref, b_ref, w_ref, y2_ref):
    dinv = _dinv_col(degp_ref[...])
    pre = dinv * (acc_ref[0] + acc_ref[1] + y_ref[...]) + b_ref[...]
    h = jnp.maximum(pre, 0.0)
    y2 = jnp.dot(h, w_ref[...], preferred_element_type=jnp.float32,
                 precision=lax.Precision.HIGHEST) * dinv
    rows = lax.broadcasted_iota(jnp.int32, (N_ROWS, D), 0)
    y2_ref[...] = jnp.where(rows < N, y2, 0.0)


def _tc_mid(acc, y, degp, b, w):
    return pl.pallas_call(
        _mid_body,
        out_shape=jax.ShapeDtypeStruct((N_ROWS, D), jnp.float32),
    )(acc, y, degp, b, w)


def _out_body(acc_ref, y_ref, degp_ref, b_ref, x_ref, wp_ref, bp_ref, o_ref):
    dinv = _dinv_col(degp_ref[...])[:N]
    pre = dinv * (acc_ref[0, :N] + acc_ref[1, :N] + y_ref[:N, :]) + b_ref[...]
    h = jnp.maximum(pre, 0.0)
    res = jnp.dot(x_ref[:N, :], wp_ref[...], preferred_element_type=jnp.float32,
                  precision=lax.Precision.HIGHEST)
    o_ref[...] = h + res + bp_ref[...]


def _tc_out(acc, y, degp, b, x, wp, bp):
    return pl.pallas_call(
        _out_body,
        out_shape=jax.ShapeDtypeStruct((N, D), jnp.float32),
    )(acc, y, degp, b, x, wp, bp)


def kernel(x, edge_index, batch, W1, b1, W2, b2, Wp, bp):
    del batch
    ept = E // NT
    ppt = K * CHUNK - ept
    pad_row = N + (jnp.arange(ppt, dtype=jnp.int32) % (N_ROWS - N))
    eidx = jnp.concatenate(
        [edge_index.astype(jnp.int32).reshape(2, NT, ept),
         jnp.broadcast_to(pad_row, (2, NT, ppt))],
        axis=2).reshape(2, NT, K, CHUNK)
    x_pad = jnp.pad(x, ((0, N_ROWS - N), (0, 0)))

    degp = _sc_degree(eidx)
    xw1 = _tc_matmul(x_pad, W1)

    y1 = _tc_scale(xw1, degp)
    acc1 = _sc_messages(y1, eidx)
    y2 = _tc_mid(acc1, y1, degp, b1.reshape(1, D), W2)
    acc2 = _sc_messages(y2, eidx)
    return _tc_out(acc2, y2, degp, b2.reshape(1, D), x_pad, Wp,
                   bp.reshape(1, D))


# speedup vs baseline: 1.0861x; 1.0861x over previous
import dataclasses
import functools

import jax
import jax.numpy as jnp
from jax import lax
from jax.experimental import pallas as pl
from jax.experimental.pallas import tpu as pltpu
from jax.experimental.pallas import tpu_sc as plsc

N = 10000
D = 128
E = 320000
NC, NS = 2, 16
NT = NC * NS
CHUNK = 112
K = 90
E_PAD = NT * K * CHUNK
IDXB = 6
NB = K // IDXB
N_ROWS = 10112
RPT = N_ROWS // NS

_mesh = plsc.VectorSubcoreMesh(core_axis_name="core", subcore_axis_name="subcore")

_sc_params = pltpu.CompilerParams(use_tc_tiling_on_sc=False)
if "needs_layout_passes" in pltpu.CompilerParams.__dataclass_fields__:
    _sc_params = dataclasses.replace(_sc_params, needs_layout_passes=False)


def _zero_rows(buf, nrows):
    z = jnp.zeros((16,), jnp.float32)

    @pl.loop(0, nrows)
    def _(r):
        for c in range(D // 16):
            buf[r, pl.ds(c * 16, 16)] = z


@functools.partial(
    pl.kernel,
    out_type=jax.ShapeDtypeStruct((NT, N_ROWS), jnp.float32),
    mesh=_mesh,
    compiler_params=_sc_params,
    scratch_types=[
        pltpu.VMEM((K, CHUNK), jnp.int32),
        pltpu.VMEM((N_ROWS,), jnp.float32),
    ],
)
def _sc_degree(eidx_hbm, deg_hbm, idx_v, hist_v):
    c = lax.axis_index("core")
    s = lax.axis_index("subcore")
    t = c * NS + s

    z = jnp.zeros((16,), jnp.float32)

    @pl.loop(0, N_ROWS // 16)
    def _(i):
        hist_v[pl.ds(i * 16, 16)] = z

    pltpu.sync_copy(eidx_hbm.at[1].at[t], idx_v)

    ones = jnp.ones((16,), jnp.float32)

    @pl.loop(0, K)
    def _(k):
        for cc in range(CHUNK // 16):
            v = idx_v[k, pl.ds(cc * 16, 16)]
            plsc.addupdate_scatter(hist_v, [v], ones)

    pltpu.sync_copy(hist_v, deg_hbm.at[t])


NBUF = 3
PF = 2
NW = -(-RPT // CHUNK)
WREM = RPT - (NW - 1) * CHUNK


@functools.partial(
    pl.kernel,
    out_type=jax.ShapeDtypeStruct((NC, N_ROWS, D), jnp.float32),
    mesh=_mesh,
    compiler_params=_sc_params,
    scratch_types=[
        pltpu.VMEM((NBUF - 1, IDXB, CHUNK), jnp.int32),
        pltpu.VMEM((NBUF - 1, IDXB, CHUNK), jnp.int32),
        pltpu.VMEM((NBUF, CHUNK, D), jnp.float32),
        pltpu.VMEM_SHARED((N_ROWS, D), jnp.float32),
        pltpu.SemaphoreType.DMA((NBUF,)),
        pltpu.SemaphoreType.DMA((NBUF,)),
        pltpu.SemaphoreType.DMA((2,)),
    ],
)
def _sc_messages(y_hbm, eidx_hbm, acc_hbm, src_v, dst_v, gbuf, acc_sh,
                 gsem, ssem, isem):
    c = lax.axis_index("core")
    s = lax.axis_index("subcore")
    t = c * NS + s
    base = s * RPT

    def iload(blk, buf):
        pltpu.async_copy(eidx_hbm.at[0].at[t].at[pl.ds(blk * IDXB, IDXB)],
                         src_v.at[buf], isem.at[buf])
        pltpu.async_copy(eidx_hbm.at[1].at[t].at[pl.ds(blk * IDXB, IDXB)],
                         dst_v.at[buf], isem.at[buf])

    def iwait(buf):
        pltpu.make_async_copy(eidx_hbm.at[0].at[t].at[pl.ds(0, IDXB)],
                              src_v.at[buf], isem.at[buf]).wait()
        pltpu.make_async_copy(eidx_hbm.at[1].at[t].at[pl.ds(0, IDXB)],
                              dst_v.at[buf], isem.at[buf]).wait()

    _zero_rows(gbuf.at[0], CHUNK)
    iload(0, 0)
    for i in range(NW):
        sz = CHUNK if i < NW - 1 else WREM
        pltpu.async_copy(gbuf.at[0].at[pl.ds(0, sz)],
                         acc_sh.at[pl.ds(base + i * CHUNK, sz)], ssem.at[0])
    for i in range(NW):
        sz = CHUNK if i < NW - 1 else WREM
        pltpu.make_async_copy(gbuf.at[0].at[pl.ds(0, sz)],
                              acc_sh.at[pl.ds(base, sz)], ssem.at[0]).wait()
    iwait(0)
    plsc.subcore_barrier()

    def gstart(b, ibuf, j):
        pltpu.async_copy(y_hbm.at[src_v.at[ibuf].at[j]], gbuf.at[b],
                         gsem.at[b])

    def gwait(b):
        pltpu.make_async_copy(y_hbm.at[src_v.at[0].at[0]], gbuf.at[b],
                              gsem.at[b]).wait()

    def sstart(b, ibuf, j):
        pltpu.async_copy(gbuf.at[b], acc_sh.at[dst_v.at[ibuf].at[j]],
                         ssem.at[b], add=True)

    def swait(b):
        pltpu.make_async_copy(gbuf.at[b], acc_sh.at[dst_v.at[0].at[0]],
                              ssem.at[b]).wait()

    gstart(0, 0, 0)
    gstart(1, 0, 1)

    @pl.loop(0, NB)
    def _(g):
        k0 = g * IDXB
        cur = g % 2
        nxt = 1 - cur
        for j in range(IDXB):
            k = k0 + j
            b = j % NBUF
            b2 = (j + PF) % NBUF

            gwait(b)
            sstart(b, cur, j)

            if j == 1:
                @pl.when(g < NB - 1)
                def _():
                    iload(g + 1, nxt)

            if j == 4:
                @pl.when(g < NB - 1)
                def _():
                    iwait(nxt)

            @pl.when(k + PF < K)
            def _():
                @pl.when(k - 1 >= 0)
                def _():
                    swait(b2)
                if j + PF < IDXB:
                    gstart(b2, cur, j + PF)
                else:
                    gstart(b2, nxt, j + PF - IDXB)

    for j in range(NBUF):
        swait((K - NBUF + j) % NBUF)

    plsc.subcore_barrier()

    for i in range(NW):
        sz = CHUNK if i < NW - 1 else WREM
        b = i % NBUF
        if i >= NBUF:
            psz = CHUNK
            pltpu.make_async_copy(gbuf.at[b].at[pl.ds(0, psz)],
                                  acc_hbm.at[c].at[pl.ds(base, psz)],
                                  ssem.at[b]).wait()
        pltpu.sync_copy(acc_sh.at[pl.ds(base + i * CHUNK, sz)],
                        gbuf.at[b].at[pl.ds(0, sz)])
        pltpu.async_copy(gbuf.at[b].at[pl.ds(0, sz)],
                         acc_hbm.at[c].at[pl.ds(base + i * CHUNK, sz)],
                         ssem.at[b])
    for j in range(NBUF):
        i = NW - NBUF + j
        sz = CHUNK if i < NW - 1 else WREM
        pltpu.make_async_copy(gbuf.at[i % NBUF].at[pl.ds(0, sz)],
                              acc_hbm.at[c].at[pl.ds(base, sz)],
                              ssem.at[i % NBUF]).wait()


_G1 = 8
_MB = N_ROWS // _G1
_G2 = 10
_OB = N // _G2


def _mm_body(x_ref, w_ref, o_ref):
    o_ref[...] = jnp.dot(x_ref[...], w_ref[...],
                         preferred_element_type=jnp.float32,
                         precision=lax.Precision.HIGHEST)


def _tc_matmul(x, w):
    return pl.pallas_call(
        _mm_body,
        out_shape=jax.ShapeDtypeStruct((N_ROWS, D), jnp.float32),
    )(x, w)


def _scale_body(xw_ref, deg_ref, y_ref):
    y_ref[...] = xw_ref[...] * lax.rsqrt(deg_ref[...])


def _tc_scale(xw, deg):
    return pl.pallas_call(
        _scale_body,
        out_shape=jax.ShapeDtypeStruct((N_ROWS, D), jnp.float32),
    )(xw, deg)


def _mid_body(acc_ref, y_ref, deg_ref, b_ref, w_ref, y2_ref):
    dinv = lax.rsqrt(deg_ref[...])
    pre = dinv * (acc_ref[0] + acc_ref[1] + y_ref[...]) + b_ref[...]
    h = jnp.maximum(pre, 0.0)
    y2 = jnp.dot(h, w_ref[...], preferred_element_type=jnp.float32,
                 precision=lax.Precision.HIGHEST) * dinv
    rows = lax.broadcasted_iota(jnp.int32, (N_ROWS, D), 0)
    y2_ref[...] = jnp.where(rows < N, y2, 0.0)


def _tc_mid(acc, y, deg, b, w):
    return pl.pallas_call(
        _mid_body,
        out_shape=jax.ShapeDtypeStruct((N_ROWS, D), jnp.float32),
    )(acc, y, deg, b, w)


def _out_body(acc_ref, y_ref, deg_ref, b_ref, x_ref, wp_ref, bp_ref, o_ref):
    dinv = lax.rsqrt(deg_ref[...])[:N]
    pre = dinv * (acc_ref[0, :N] + acc_ref[1, :N] + y_ref[:N, :]) + b_ref[...]
    h = jnp.maximum(pre, 0.0)
    res = jnp.dot(x_ref[:N, :], wp_ref[...], preferred_element_type=jnp.float32,
                  precision=lax.Precision.HIGHEST)
    o_ref[...] = h + res + bp_ref[...]


def _tc_out(acc, y, deg, b, x, wp, bp):
    return pl.pallas_call(
        _out_body,
        out_shape=jax.ShapeDtypeStruct((N, D), jnp.float32),
    )(acc, y, deg, b, x, wp, bp)


def kernel(x, edge_index, batch, W1, b1, W2, b2, Wp, bp):
    del batch
    ept = E // NT
    ppt = K * CHUNK - ept
    pad_row = N + (jnp.arange(ppt, dtype=jnp.int32) % (N_ROWS - N))
    eidx = jnp.concatenate(
        [edge_index.astype(jnp.int32).reshape(2, NT, ept),
         jnp.broadcast_to(pad_row, (2, NT, ppt))],
        axis=2).reshape(2, NT, K, CHUNK)
    x_pad = jnp.pad(x, ((0, N_ROWS - N), (0, 0)))

    degp = _sc_degree(eidx)
    xw1 = _tc_matmul(x_pad, W1)
    deg = (degp.sum(axis=0) + 1.0).reshape(N_ROWS, 1)

    y1 = _tc_scale(xw1, deg)
    acc1 = _sc_messages(y1, eidx)
    y2 = _tc_mid(acc1, y1, deg, b1.reshape(1, D), W2)
    acc2 = _sc_messages(y2, eidx)
    return _tc_out(acc2, y2, deg, b2.reshape(1, D), x_pad, Wp,
                   bp.reshape(1, D))
